# baseline (speedup 1.0000x reference)
import functools

import jax
import jax.numpy as jnp
from jax import lax
from jax.experimental import pallas as pl
from jax.experimental.pallas import tpu as pltpu
from jax.experimental.pallas import tpu_sc as plsc

N = 10000
E = 320000
NFEAT = 128
NHID = 16
NHEADS = 8
NCLASS = 16
NSTRUC = 64
ALPHA = 0.2

NC = 2
NS = 16
NW = NC * NS
CH = 80
NCHUNK = 126
EPT = NCHUNK * CH
E_PAD = NW * EPT
GRP = 18
NGROUP = NCHUNK // GRP
N_PAD = 10016
ROWS_PER_SUB = N_PAD // NS
MW = NHEADS * NHID + 16
MW2 = 2 * NCLASS

_f32 = jnp.float32
_mesh = plsc.VectorSubcoreMesh(core_axis_name="c", subcore_axis_name="s",
                               num_cores=NC, num_subcores=NS)

_GDN = lax.GatherDimensionNumbers(offset_dims=(), collapsed_slice_dims=(0,),
                                  start_index_map=(0,))


def _lane_splat(v, lane):
    return lax.gather(v, jnp.full((16, 1), lane, jnp.int32), _GDN, (1,),
                      mode=lax.GatherScatterMode.PROMISE_IN_BOUNDS)


def _tca_body(x_ref, wcat_ref, a1p_ref, a2p_ref, hx_ref, sa_ref):
    h = jnp.dot(x_ref[...], wcat_ref[...], preferred_element_type=_f32)
    sb = jnp.dot(h, a2p_ref[...], preferred_element_type=_f32)
    hx_ref[...] = jnp.concatenate([h, sb], axis=1)
    sa_ref[...] = jnp.dot(h, a1p_ref[...], preferred_element_type=_f32)


def _tcb_body(p0_ref, p1_ref, exp_ref, wout_ref, a2c_ref, e1w_ref, e1b_ref,
              outh_ref, t_ref, y_ref):
    acc = p0_ref[...] + p1_ref[...]
    feats = acc[:, :NHEADS * NHID]
    rs = acc[:, NHEADS * NHID:NHEADS * NHID + NHEADS]
    rinv = 1.0 / (rs + 1e-16)
    hp = feats * jnp.dot(rinv, exp_ref[...], preferred_element_type=_f32)
    h1 = jnp.where(hp > 0, hp, jnp.exp(hp) - 1.0)
    outh = jnp.dot(h1, wout_ref[...], preferred_element_type=_f32)
    outh_ref[...] = outh
    t_ref[...] = jnp.dot(outh, a2c_ref[...], preferred_element_type=_f32)
    y_ref[...] = jnp.dot(h1, e1w_ref[...], preferred_element_type=_f32) + e1b_ref[...]


def _tcc_body(q0_ref, q1_ref, e2w_ref, e2b_ref, xlog_ref, z_ref):
    acc = q0_ref[...] + q1_ref[...]
    out = acc[:, :NCLASS] / (acc[:, NCLASS:] + 1e-16)
    z_ref[...] = jnp.dot(out, e2w_ref[...], preferred_element_type=_f32) + e2b_ref[...]
    eo = jnp.where(out > 0, out, jnp.exp(out) - 1.0)
    m = jnp.max(eo, axis=1, keepdims=True)
    lse = jnp.log(jnp.sum(jnp.exp(eo - m), axis=1, keepdims=True)) + m
    xlog_ref[...] = eo - lse


@functools.partial(
    pl.kernel,
    out_type=jax.ShapeDtypeStruct((NC, N_PAD, MW), _f32),
    mesh=_mesh,
    compiler_params=pltpu.CompilerParams(use_tc_tiling_on_sc=False,
                                         needs_layout_passes=False),
    scratch_types=[
        pltpu.VMEM((GRP, CH), jnp.int32),
        pltpu.VMEM((GRP, CH), jnp.int32),
        pltpu.VMEM((CH, 16), _f32),
        pltpu.VMEM((CH, 16), _f32),
        pltpu.VMEM((CH, MW), _f32),
        pltpu.VMEM((CH, MW), _f32),
        pltpu.VMEM((CH, MW), _f32),
        pltpu.VMEM_SHARED((N_PAD, MW), _f32),
        pltpu.SemaphoreType.DMA,
        pltpu.SemaphoreType.DMA,
        pltpu.SemaphoreType.DMA,
        pltpu.SemaphoreType.DMA,
        pltpu.SemaphoreType.DMA,
        pltpu.SemaphoreType.DMA,
        pltpu.SemaphoreType.DMA,
        pltpu.SemaphoreType.DMA,
    ],
)
def _sc_layer1(src2_hbm, dst2_hbm, sa_hbm, hx_hbm, out_hbm,
               sidx, didx, sag0, sag1, msg0, msg1, msg2, acc,
               si_sem, di_sem, sa_sem0, sa_sem1, mg_sem0, mg_sem1, mg_sem2,
               sc_sem):
    cid = lax.axis_index("c")
    sid = lax.axis_index("s")
    wid = sid * NC + cid
    brow = wid * NCHUNK
    zero16 = jnp.zeros((16,), _f32)
    sags = (sag0, sag1)
    msgs = (msg0, msg1, msg2)
    sasems = (sa_sem0, sa_sem1)
    mgsems = (mg_sem0, mg_sem1, mg_sem2)

    def zrow(r, carry):
        for cth in range(MW // 16):
            msg0[r, pl.ds(cth * 16, 16)] = zero16
        return carry

    lax.fori_loop(0, CH, zrow, 0)
    for k in range(ROWS_PER_SUB // CH):
        pltpu.sync_copy(msg0, acc.at[pl.ds(sid * ROWS_PER_SUB + k * CH, CH)])
    if ROWS_PER_SUB % CH:
        pltpu.sync_copy(
            msg0.at[pl.ds(0, ROWS_PER_SUB % CH)],
            acc.at[pl.ds(sid * ROWS_PER_SUB + (ROWS_PER_SUB // CH) * CH,
                         ROWS_PER_SUB % CH)])
    plsc.subcore_barrier()

    def group_body(g, carry):
        r0 = brow + g * GRP
        hsi = pltpu.async_copy(src2_hbm.at[pl.ds(r0, GRP)], sidx, si_sem)
        hdi = pltpu.async_copy(dst2_hbm.at[pl.ds(r0, GRP)], didx, di_sem)
        hsi.wait()
        hdi.wait()

        def issue(j):
            ha = pltpu.async_copy(sa_hbm.at[sidx.at[j]], sags[j % 2],
                                  sasems[j % 2])
            hb = pltpu.async_copy(hx_hbm.at[didx.at[j]], msgs[j % 3],
                                  mgsems[j % 3])
            return ha, hb

        pend = issue(0)
        pend_sc = None
        for j in range(GRP):
            nxt = issue(j + 1) if j + 1 < GRP else None
            pend[0].wait()
            pend[1].wait()
            sag_s = sags[j % 2]
            msg_s = msgs[j % 3]

            def edge_body(e4, ecarry):
                es = [e4 * 8 + k for k in range(8)]
                ts = [sag_s[e, :] + msg_s[e, pl.ds(NHEADS * NHID, 16)]
                      for e in es]
                ees = [jnp.exp(-jnp.maximum(t, ALPHA * t)) for t in ts]
                for e, ee in zip(es, ees):
                    msg_s[e, pl.ds(NHEADS * NHID, 16)] = ee
                for h in range(NHEADS):
                    for e, ee in zip(es, ees):
                        eh = _lane_splat(ee, h)
                        msg_s[e, pl.ds(h * NHID, NHID)] = (
                            eh * msg_s[e, pl.ds(h * NHID, NHID)])
                return ecarry

            lax.fori_loop(0, CH // 8, edge_body, 0)
            if pend_sc is not None:
                pend_sc.wait()
            if j < GRP - 1:
                pend_sc = pltpu.async_copy(msg_s, acc.at[sidx.at[j]], sc_sem,
                                           add=True)
            else:
                pltpu.sync_copy(msg_s, acc.at[sidx.at[j]], add=True)
            pend = nxt
        return carry

    lax.fori_loop(0, NGROUP, group_body, 0)
    plsc.subcore_barrier()
    pltpu.sync_copy(acc.at[pl.ds(sid * ROWS_PER_SUB, ROWS_PER_SUB)],
                    out_hbm.at[cid, pl.ds(sid * ROWS_PER_SUB, ROWS_PER_SUB)])


@functools.partial(
    pl.kernel,
    out_type=jax.ShapeDtypeStruct((NC, N_PAD, MW2), _f32),
    mesh=_mesh,
    compiler_params=pltpu.CompilerParams(use_tc_tiling_on_sc=False,
                                         needs_layout_passes=False),
    scratch_types=[
        pltpu.VMEM((GRP, CH), jnp.int32),
        pltpu.VMEM((GRP, CH), jnp.int32),
        pltpu.VMEM((N_PAD,), _f32),
        pltpu.VMEM((N_PAD,), _f32),
        pltpu.VMEM((CH, NCLASS), _f32),
        pltpu.VMEM((CH, NCLASS), _f32),
        pltpu.VMEM((CH, MW2), _f32),
        pltpu.VMEM((CH, MW2), _f32),
        pltpu.VMEM((CH, MW2), _f32),
        pltpu.VMEM_SHARED((N_PAD, MW2), _f32),
        pltpu.SemaphoreType.DMA,
        pltpu.SemaphoreType.DMA,
        pltpu.SemaphoreType.DMA,
        pltpu.SemaphoreType.DMA,
        pltpu.SemaphoreType.DMA,
    ],
)
def _sc_layer2(src2_hbm, dst2_hbm, t1_hbm, t2_hbm, th_hbm, out_hbm,
               sidx, didx, t1v, t2v, thg0, thg1, msg0, msg1, msg2, acc,
               si_sem, di_sem, th_sem0, th_sem1, sc_sem):
    cid = lax.axis_index("c")
    sid = lax.axis_index("s")
    wid = sid * NC + cid
    brow = wid * NCHUNK
    zero16 = jnp.zeros((16,), _f32)
    thgs = (thg0, thg1)
    msgs = (msg0, msg1, msg2)
    thsems = (th_sem0, th_sem1)

    pltpu.sync_copy(t1_hbm, t1v)
    pltpu.sync_copy(t2_hbm, t2v)

    def zrow(r, carry):
        msg0[r, pl.ds(0, 16)] = zero16
        msg0[r, pl.ds(16, 16)] = zero16
        return carry

    lax.fori_loop(0, CH, zrow, 0)
    for k in range(ROWS_PER_SUB // CH):
        pltpu.sync_copy(msg0, acc.at[pl.ds(sid * ROWS_PER_SUB + k * CH, CH)])
    if ROWS_PER_SUB % CH:
        pltpu.sync_copy(
            msg0.at[pl.ds(0, ROWS_PER_SUB % CH)],
            acc.at[pl.ds(sid * ROWS_PER_SUB + (ROWS_PER_SUB // CH) * CH,
                         ROWS_PER_SUB % CH)])
    plsc.subcore_barrier()

    def group_body(g, carry):
        r0 = brow + g * GRP
        hsi = pltpu.async_copy(src2_hbm.at[pl.ds(r0, GRP)], sidx, si_sem)
        hdi = pltpu.async_copy(dst2_hbm.at[pl.ds(r0, GRP)], didx, di_sem)
        hsi.wait()
        hdi.wait()

        def issue(j):
            s = j % 2
            return pltpu.async_copy(th_hbm.at[didx.at[j]], thgs[s], thsems[s])

        pend = issue(0)
        pend_sc = None
        for j in range(GRP):
            nxt = issue(j + 1) if j + 1 < GRP else None
            pend.wait()
            thg_s = thgs[j % 2]
            msg_s = msgs[j % 3]
            for gg in range(CH // 16):
                s16 = sidx[j, pl.ds(gg * 16, 16)]
                d16 = didx[j, pl.ds(gg * 16, 16)]
                v1 = plsc.load_gather(t1v, [s16])
                v2 = plsc.load_gather(t2v, [d16])
                t = v1 + v2
                ee = jnp.exp(-jnp.maximum(t, ALPHA * t))
                for e2 in range(16):
                    eh = _lane_splat(ee, e2)
                    row = gg * 16 + e2
                    msg_s[row, pl.ds(0, 16)] = eh * thg_s[row, :]
                    msg_s[row, pl.ds(16, 16)] = eh
            if pend_sc is not None:
                pend_sc.wait()
            if j < GRP - 1:
                pend_sc = pltpu.async_copy(msg_s, acc.at[sidx.at[j]], sc_sem,
                                           add=True)
            else:
                pltpu.sync_copy(msg_s, acc.at[sidx.at[j]], add=True)
            pend = nxt
        return carry

    lax.fori_loop(0, NGROUP, group_body, 0)
    plsc.subcore_barrier()
    pltpu.sync_copy(acc.at[pl.ds(sid * ROWS_PER_SUB, ROWS_PER_SUB)],
                    out_hbm.at[cid, pl.ds(sid * ROWS_PER_SUB, ROWS_PER_SUB)])


def kernel(x, edge_index, W, A, W_out, A_out, enc1_W, enc1_b, enc2_W, enc2_b):
    pe = E_PAD - E
    srcp = jnp.concatenate([edge_index[0], jnp.full((pe,), N, jnp.int32)])
    dstp = jnp.concatenate([edge_index[1], jnp.full((pe,), N, jnp.int32)])
    src2 = srcp.reshape(E_PAD // CH, CH)
    dst2 = dstp.reshape(E_PAD // CH, CH)
    xp = jnp.pad(x, ((0, N_PAD - N), (0, 0)))

    Wcat = W.transpose(1, 0, 2).reshape(NFEAT, NHEADS * NHID)
    eye8 = jnp.eye(NHEADS, dtype=_f32)
    A1 = A[:, :NHID]
    A2 = A[:, NHID:]
    A1p = jnp.pad((A1[:, :, None] * eye8[:, None, :]).reshape(NHEADS * NHID, NHEADS),
                  ((0, 0), (0, 16 - NHEADS)))
    A2p = jnp.pad((A2[:, :, None] * eye8[:, None, :]).reshape(NHEADS * NHID, NHEADS),
                  ((0, 0), (0, 16 - NHEADS)))
    EXPAND = jnp.kron(eye8, jnp.ones((1, NHID), _f32))
    A2c = jnp.pad(jnp.stack([A_out[:NCLASS], A_out[NCLASS:]], axis=1),
                  ((0, 0), (0, 14)))

    Hx, Sa = pl.pallas_call(
        _tca_body,
        out_shape=[
            jax.ShapeDtypeStruct((N_PAD, MW), _f32),
            jax.ShapeDtypeStruct((N_PAD, 16), _f32),
        ],
    )(xp, Wcat, A1p, A2p)

    parts = _sc_layer1(src2, dst2, Sa, Hx)

    outh, T, y = pl.pallas_call(
        _tcb_body,
        out_shape=[
            jax.ShapeDtypeStruct((N_PAD, NCLASS), _f32),
            jax.ShapeDtypeStruct((N_PAD, 16), _f32),
            jax.ShapeDtypeStruct((N_PAD, NSTRUC), _f32),
        ],
    )(parts[0], parts[1], EXPAND, W_out, A2c, enc1_W, enc1_b.reshape(1, -1))

    parts2 = _sc_layer2(src2, dst2, T[:, 0], T[:, 1], outh)

    xlog, z = pl.pallas_call(
        _tcc_body,
        out_shape=[
            jax.ShapeDtypeStruct((N_PAD, NCLASS), _f32),
            jax.ShapeDtypeStruct((N_PAD, NSTRUC), _f32),
        ],
    )(parts2[0], parts2[1], enc2_W, enc2_b.reshape(1, -1))

    return (xlog[:N], y[:N], z[:N])


# speedup vs baseline: 1.1247x; 1.1247x over previous
import functools

import jax
import jax.numpy as jnp
from jax import lax
from jax.experimental import pallas as pl
from jax.experimental.pallas import tpu as pltpu
from jax.experimental.pallas import tpu_sc as plsc

N = 10000
E = 320000
NFEAT = 128
NHID = 16
NHEADS = 8
NCLASS = 16
NSTRUC = 64
ALPHA = 0.2

NC = 2
NS = 16
NW = NC * NS
CH = 80
EPT = 10000
E_PAD = NW * EPT
NCHUNK = EPT // CH
GRP = 5
NGROUP = NCHUNK // GRP
N_PAD = 10016
ROWS_PER_SUB = N_PAD // NS
MW = NHEADS * NHID + 16
MW2 = 2 * NCLASS

_f32 = jnp.float32
_mesh = plsc.VectorSubcoreMesh(core_axis_name="c", subcore_axis_name="s",
                               num_cores=NC, num_subcores=NS)

_GDN = lax.GatherDimensionNumbers(offset_dims=(), collapsed_slice_dims=(0,),
                                  start_index_map=(0,))


def _lane_splat(v, lane):
    return lax.gather(v, jnp.full((16, 1), lane, jnp.int32), _GDN, (1,),
                      mode=lax.GatherScatterMode.PROMISE_IN_BOUNDS)


def _tca_body(x_ref, wcat_ref, a1p_ref, a2p_ref, hx_ref, sa_ref):
    h = jnp.dot(x_ref[...], wcat_ref[...], preferred_element_type=_f32)
    sb = jnp.dot(h, a2p_ref[...], preferred_element_type=_f32)
    hx_ref[...] = jnp.concatenate([h, sb], axis=1)
    sa_ref[...] = jnp.dot(h, a1p_ref[...], preferred_element_type=_f32)


def _tcb_body(p0_ref, p1_ref, exp_ref, wout_ref, a2c_ref, e1w_ref, e1b_ref,
              outh_ref, t_ref, y_ref):
    acc = p0_ref[...] + p1_ref[...]
    feats = acc[:, :NHEADS * NHID]
    rs = acc[:, NHEADS * NHID:NHEADS * NHID + NHEADS]
    rinv = 1.0 / (rs + 1e-16)
    hp = feats * jnp.dot(rinv, exp_ref[...], preferred_element_type=_f32)
    h1 = jnp.where(hp > 0, hp, jnp.exp(hp) - 1.0)
    outh = jnp.dot(h1, wout_ref[...], preferred_element_type=_f32)
    outh_ref[...] = outh
    t_ref[...] = jnp.dot(outh, a2c_ref[...], preferred_element_type=_f32)
    y_ref[...] = jnp.dot(h1, e1w_ref[...], preferred_element_type=_f32) + e1b_ref[...]


def _tcc_body(q0_ref, q1_ref, e2w_ref, e2b_ref, xlog_ref, z_ref):
    acc = q0_ref[...] + q1_ref[...]
    out = acc[:, :NCLASS] / (acc[:, NCLASS:] + 1e-16)
    z_ref[...] = jnp.dot(out, e2w_ref[...], preferred_element_type=_f32) + e2b_ref[...]
    eo = jnp.where(out > 0, out, jnp.exp(out) - 1.0)
    m = jnp.max(eo, axis=1, keepdims=True)
    lse = jnp.log(jnp.sum(jnp.exp(eo - m), axis=1, keepdims=True)) + m
    xlog_ref[...] = eo - lse


@functools.partial(
    pl.kernel,
    out_type=jax.ShapeDtypeStruct((NC, N_PAD, MW), _f32),
    mesh=_mesh,
    compiler_params=pltpu.CompilerParams(use_tc_tiling_on_sc=False,
                                         needs_layout_passes=False),
    scratch_types=[
        pltpu.VMEM((GRP, CH), jnp.int32),
        pltpu.VMEM((GRP, CH), jnp.int32),
        pltpu.VMEM((CH, 16), _f32),
        pltpu.VMEM((CH, 16), _f32),
        pltpu.VMEM((CH, MW), _f32),
        pltpu.VMEM((CH, MW), _f32),
        pltpu.VMEM((CH, MW), _f32),
        pltpu.VMEM_SHARED((N_PAD, MW), _f32),
        pltpu.SemaphoreType.DMA,
        pltpu.SemaphoreType.DMA,
        pltpu.SemaphoreType.DMA,
        pltpu.SemaphoreType.DMA,
        pltpu.SemaphoreType.DMA,
        pltpu.SemaphoreType.DMA,
        pltpu.SemaphoreType.DMA,
        pltpu.SemaphoreType.DMA,
    ],
)
def _sc_layer1(src2_hbm, dst2_hbm, sa_hbm, hx_hbm, out_hbm,
               sidx, didx, sag0, sag1, msg0, msg1, msg2, acc,
               si_sem, di_sem, sa_sem0, sa_sem1, mg_sem0, mg_sem1, mg_sem2,
               sc_sem):
    cid = lax.axis_index("c")
    sid = lax.axis_index("s")
    wid = sid * NC + cid
    brow = wid * NCHUNK
    zero16 = jnp.zeros((16,), _f32)
    sags = (sag0, sag1)
    msgs = (msg0, msg1, msg2)
    sasems = (sa_sem0, sa_sem1)
    mgsems = (mg_sem0, mg_sem1, mg_sem2)

    def zrow(r, carry):
        for cth in range(MW // 16):
            msg0[r, pl.ds(cth * 16, 16)] = zero16
        return carry

    lax.fori_loop(0, CH, zrow, 0)
    for k in range(ROWS_PER_SUB // CH):
        pltpu.sync_copy(msg0, acc.at[pl.ds(sid * ROWS_PER_SUB + k * CH, CH)])
    if ROWS_PER_SUB % CH:
        pltpu.sync_copy(
            msg0.at[pl.ds(0, ROWS_PER_SUB % CH)],
            acc.at[pl.ds(sid * ROWS_PER_SUB + (ROWS_PER_SUB // CH) * CH,
                         ROWS_PER_SUB % CH)])
    plsc.subcore_barrier()

    def group_body(g, carry):
        r0 = brow + g * GRP
        hsi = pltpu.async_copy(src2_hbm.at[pl.ds(r0, GRP)], sidx, si_sem)
        hdi = pltpu.async_copy(dst2_hbm.at[pl.ds(r0, GRP)], didx, di_sem)
        hsi.wait()
        hdi.wait()

        def issue(j):
            ha = pltpu.async_copy(sa_hbm.at[sidx.at[j]], sags[j % 2],
                                  sasems[j % 2])
            hb = pltpu.async_copy(hx_hbm.at[didx.at[j]], msgs[j % 3],
                                  mgsems[j % 3])
            return ha, hb

        pend = issue(0)
        pend_sc = None
        for j in range(GRP):
            nxt = issue(j + 1) if j + 1 < GRP else None
            pend[0].wait()
            pend[1].wait()
            sag_s = sags[j % 2]
            msg_s = msgs[j % 3]

            def edge_body(e4, ecarry):
                es = [e4 * 8 + k for k in range(8)]
                ts = [sag_s[e, :] + msg_s[e, pl.ds(NHEADS * NHID, 16)]
                      for e in es]
                ees = [jnp.exp(-jnp.maximum(t, ALPHA * t)) for t in ts]
                for e, ee in zip(es, ees):
                    msg_s[e, pl.ds(NHEADS * NHID, 16)] = ee
                for h in range(NHEADS):
                    for e, ee in zip(es, ees):
                        eh = _lane_splat(ee, h)
                        msg_s[e, pl.ds(h * NHID, NHID)] = (
                            eh * msg_s[e, pl.ds(h * NHID, NHID)])
                return ecarry

            lax.fori_loop(0, CH // 8, edge_body, 0)
            if pend_sc is not None:
                pend_sc.wait()
            if j < GRP - 1:
                pend_sc = pltpu.async_copy(msg_s, acc.at[sidx.at[j]], sc_sem,
                                           add=True)
            else:
                pltpu.sync_copy(msg_s, acc.at[sidx.at[j]], add=True)
            pend = nxt
        return carry

    lax.fori_loop(0, NGROUP, group_body, 0)
    plsc.subcore_barrier()
    pltpu.sync_copy(acc.at[pl.ds(sid * ROWS_PER_SUB, ROWS_PER_SUB)],
                    out_hbm.at[cid, pl.ds(sid * ROWS_PER_SUB, ROWS_PER_SUB)])


@functools.partial(
    pl.kernel,
    out_type=jax.ShapeDtypeStruct((NC, N_PAD, MW2), _f32),
    mesh=_mesh,
    compiler_params=pltpu.CompilerParams(use_tc_tiling_on_sc=False,
                                         needs_layout_passes=False),
    scratch_types=[
        pltpu.VMEM((GRP, CH), jnp.int32),
        pltpu.VMEM((GRP, CH), jnp.int32),
        pltpu.VMEM((N_PAD,), _f32),
        pltpu.VMEM((N_PAD,), _f32),
        pltpu.VMEM((CH, NCLASS), _f32),
        pltpu.VMEM((CH, NCLASS), _f32),
        pltpu.VMEM((CH, MW2), _f32),
        pltpu.VMEM((CH, MW2), _f32),
        pltpu.VMEM((CH, MW2), _f32),
        pltpu.VMEM_SHARED((N_PAD, MW2), _f32),
        pltpu.SemaphoreType.DMA,
        pltpu.SemaphoreType.DMA,
        pltpu.SemaphoreType.DMA,
        pltpu.SemaphoreType.DMA,
        pltpu.SemaphoreType.DMA,
    ],
)
def _sc_layer2(src2_hbm, dst2_hbm, t1_hbm, t2_hbm, th_hbm, out_hbm,
               sidx, didx, t1v, t2v, thg0, thg1, msg0, msg1, msg2, acc,
               si_sem, di_sem, th_sem0, th_sem1, sc_sem):
    cid = lax.axis_index("c")
    sid = lax.axis_index("s")
    wid = sid * NC + cid
    brow = wid * NCHUNK
    zero16 = jnp.zeros((16,), _f32)
    thgs = (thg0, thg1)
    msgs = (msg0, msg1, msg2)
    thsems = (th_sem0, th_sem1)

    pltpu.sync_copy(t1_hbm, t1v)
    pltpu.sync_copy(t2_hbm, t2v)

    def zrow(r, carry):
        msg0[r, pl.ds(0, 16)] = zero16
        msg0[r, pl.ds(16, 16)] = zero16
        return carry

    lax.fori_loop(0, CH, zrow, 0)
    for k in range(ROWS_PER_SUB // CH):
        pltpu.sync_copy(msg0, acc.at[pl.ds(sid * ROWS_PER_SUB + k * CH, CH)])
    if ROWS_PER_SUB % CH:
        pltpu.sync_copy(
            msg0.at[pl.ds(0, ROWS_PER_SUB % CH)],
            acc.at[pl.ds(sid * ROWS_PER_SUB + (ROWS_PER_SUB // CH) * CH,
                         ROWS_PER_SUB % CH)])
    plsc.subcore_barrier()

    def group_body(g, carry):
        r0 = brow + g * GRP
        hsi = pltpu.async_copy(src2_hbm.at[pl.ds(r0, GRP)], sidx, si_sem)
        hdi = pltpu.async_copy(dst2_hbm.at[pl.ds(r0, GRP)], didx, di_sem)
        hsi.wait()
        hdi.wait()

        def issue(j):
            s = j % 2
            return pltpu.async_copy(th_hbm.at[didx.at[j]], thgs[s], thsems[s])

        pend = issue(0)
        pend_sc = None
        for j in range(GRP):
            nxt = issue(j + 1) if j + 1 < GRP else None
            pend.wait()
            thg_s = thgs[j % 2]
            msg_s = msgs[j % 3]
            for gg in range(CH // 16):
                s16 = sidx[j, pl.ds(gg * 16, 16)]
                d16 = didx[j, pl.ds(gg * 16, 16)]
                v1 = plsc.load_gather(t1v, [s16])
                v2 = plsc.load_gather(t2v, [d16])
                t = v1 + v2
                ee = jnp.exp(-jnp.maximum(t, ALPHA * t))
                for e2 in range(16):
                    eh = _lane_splat(ee, e2)
                    row = gg * 16 + e2
                    msg_s[row, pl.ds(0, 16)] = eh * thg_s[row, :]
                    msg_s[row, pl.ds(16, 16)] = eh
            if pend_sc is not None:
                pend_sc.wait()
            if j < GRP - 1:
                pend_sc = pltpu.async_copy(msg_s, acc.at[sidx.at[j]], sc_sem,
                                           add=True)
            else:
                pltpu.sync_copy(msg_s, acc.at[sidx.at[j]], add=True)
            pend = nxt
        return carry

    lax.fori_loop(0, NGROUP, group_body, 0)
    plsc.subcore_barrier()
    pltpu.sync_copy(acc.at[pl.ds(sid * ROWS_PER_SUB, ROWS_PER_SUB)],
                    out_hbm.at[cid, pl.ds(sid * ROWS_PER_SUB, ROWS_PER_SUB)])


def kernel(x, edge_index, W, A, W_out, A_out, enc1_W, enc1_b, enc2_W, enc2_b):
    src2 = edge_index[0].reshape(E_PAD // CH, CH)
    dst2 = edge_index[1].reshape(E_PAD // CH, CH)
    xp = jnp.pad(x, ((0, N_PAD - N), (0, 0)))

    Wcat = W.transpose(1, 0, 2).reshape(NFEAT, NHEADS * NHID)
    eye8 = jnp.eye(NHEADS, dtype=_f32)
    A1 = A[:, :NHID]
    A2 = A[:, NHID:]
    A1p = jnp.pad((A1[:, :, None] * eye8[:, None, :]).reshape(NHEADS * NHID, NHEADS),
                  ((0, 0), (0, 16 - NHEADS)))
    A2p = jnp.pad((A2[:, :, None] * eye8[:, None, :]).reshape(NHEADS * NHID, NHEADS),
                  ((0, 0), (0, 16 - NHEADS)))
    EXPAND = jnp.kron(eye8, jnp.ones((1, NHID), _f32))
    A2c = jnp.pad(jnp.stack([A_out[:NCLASS], A_out[NCLASS:]], axis=1),
                  ((0, 0), (0, 14)))

    Hx, Sa = pl.pallas_call(
        _tca_body,
        out_shape=[
            jax.ShapeDtypeStruct((N_PAD, MW), _f32),
            jax.ShapeDtypeStruct((N_PAD, 16), _f32),
        ],
    )(xp, Wcat, A1p, A2p)

    parts = _sc_layer1(src2, dst2, Sa, Hx)

    outh, T, y = pl.pallas_call(
        _tcb_body,
        out_shape=[
            jax.ShapeDtypeStruct((N_PAD, NCLASS), _f32),
            jax.ShapeDtypeStruct((N_PAD, 16), _f32),
            jax.ShapeDtypeStruct((N_PAD, NSTRUC), _f32),
        ],
    )(parts[0], parts[1], EXPAND, W_out, A2c, enc1_W, enc1_b.reshape(1, -1))

    parts2 = _sc_layer2(src2, dst2, T[:, 0], T[:, 1], outh)

    xlog, z = pl.pallas_call(
        _tcc_body,
        out_shape=[
            jax.ShapeDtypeStruct((N_PAD, NCLASS), _f32),
            jax.ShapeDtypeStruct((N_PAD, NSTRUC), _f32),
        ],
    )(parts2[0], parts2[1], enc2_W, enc2_b.reshape(1, -1))

    return (xlog[:N], y[:N], z[:N])
